# baseline (speedup 1.0000x reference)
import functools

import jax
import jax.numpy as jnp
from jax import lax
from jax.experimental import pallas as pl
from jax.experimental.pallas import tpu as pltpu
from jax.experimental.pallas import tpu_sc as plsc

IN_CHANNELS = 384
N_EXPERTS = 64
TOP_K = 8
ROUTE_SCALE = 1.0

B = 32
H = 56
W_SP = 56
SPATIAL = H * W_SP

TC_H = 42
BATCH_BLK = 8
H_BLK = 14
N_BATCH_BLKS = B // BATCH_BLK
N_H_BLKS = TC_H // H_BLK

SC_H = H - TC_H
SC_CHUNK = 2
SC_STEPS = SC_H // SC_CHUNK
N_CREG = IN_CHANNELS // 16


def _tc_pool_kernel(x_ref, out_ref):
    hi = pl.program_id(1)
    part = jnp.sum(x_ref[...], axis=(1, 2))

    @pl.when(hi == 0)
    def _init():
        out_ref[...] = part

    @pl.when(hi != 0)
    def _accum():
        out_ref[...] += part


_sc_mesh = plsc.VectorSubcoreMesh(core_axis_name="c", subcore_axis_name="s")


@functools.partial(
    pl.kernel,
    mesh=_sc_mesh,
    out_type=jax.ShapeDtypeStruct((B, IN_CHANNELS), jnp.float32),
    scratch_types=[
        pltpu.VMEM((SC_CHUNK, W_SP, IN_CHANNELS), jnp.float32),
        pltpu.VMEM((SC_CHUNK, W_SP, IN_CHANNELS), jnp.float32),
        pltpu.VMEM((IN_CHANNELS,), jnp.float32),
        pltpu.SemaphoreType.DMA,
        pltpu.SemaphoreType.DMA,
    ],
)
def _sc_pool_kernel(x_hbm, out_hbm, buf0, buf1, accv, sem0, sem1):
    b = lax.axis_index("s") * 2 + lax.axis_index("c")
    bufs = (buf0, buf1)
    sems = (sem0, sem1)

    pltpu.async_copy(x_hbm.at[b, pl.ds(TC_H, SC_CHUNK)], buf0, sem0)

    accs = tuple(jnp.zeros((16,), jnp.float32) for _ in range(N_CREG))
    for t in range(SC_STEPS):
        if t + 1 < SC_STEPS:
            pltpu.async_copy(
                x_hbm.at[b, pl.ds(TC_H + (t + 1) * SC_CHUNK, SC_CHUNK)],
                bufs[(t + 1) % 2],
                sems[(t + 1) % 2],
            )
        pltpu.make_async_copy(
            x_hbm.at[b, pl.ds(TC_H + t * SC_CHUNK, SC_CHUNK)],
            bufs[t % 2],
            sems[t % 2],
        ).wait()
        buf = bufs[t % 2]
        for tt in range(SC_CHUNK):

            def row_body(i, accs, _buf=buf, _tt=tt):
                return tuple(
                    accs[j] + _buf[_tt, i, pl.ds(16 * j, 16)]
                    for j in range(N_CREG)
                )

            accs = lax.fori_loop(0, W_SP, row_body, accs)

    for j in range(N_CREG):
        accv[pl.ds(16 * j, 16)] = accs[j]
    pltpu.sync_copy(accv, out_hbm.at[b])


def _router_kernel(p_ref, q_ref, w_ref, b_ref, bias_ref, wout_ref, iout_ref):
    pooled = (p_ref[...] + q_ref[...]) * (1.0 / SPATIAL)
    logits = jax.lax.dot_general(
        pooled,
        w_ref[...],
        (((1,), (1,)), ((), ())),
        preferred_element_type=jnp.float32,
    ) + b_ref[...]
    scores = jax.nn.sigmoid(logits)
    s = scores + bias_ref[...]

    iota = jax.lax.broadcasted_iota(jnp.int32, (B, N_EXPERTS), 1)
    idx_cols = []
    w_cols = []
    for _ in range(TOP_K):
        m = jnp.max(s, axis=1, keepdims=True)
        idx = jnp.min(
            jnp.where(s == m, iota, N_EXPERTS), axis=1, keepdims=True
        )
        onehot = iota == idx
        w = jnp.sum(jnp.where(onehot, scores, 0.0), axis=1, keepdims=True)
        idx_cols.append(idx)
        w_cols.append(w)
        s = jnp.where(onehot, -jnp.inf, s)
    indices = jnp.concatenate(idx_cols, axis=1)
    weights = jnp.concatenate(w_cols, axis=1)
    weights = weights / jnp.sum(weights, axis=1, keepdims=True)
    wout_ref[...] = weights * ROUTE_SCALE
    iout_ref[...] = indices


@jax.jit
def kernel(x, W, b, bias_buf):
    xt = x.transpose(0, 2, 3, 1)

    sums_sc = _sc_pool_kernel(xt)

    sums_tc = pl.pallas_call(
        _tc_pool_kernel,
        grid=(N_BATCH_BLKS, N_H_BLKS),
        in_specs=[
            pl.BlockSpec(
                (BATCH_BLK, H_BLK, W_SP, IN_CHANNELS), lambda bi, hi: (bi, hi, 0, 0)
            ),
        ],
        out_specs=pl.BlockSpec((BATCH_BLK, IN_CHANNELS), lambda bi, hi: (bi, 0)),
        out_shape=jax.ShapeDtypeStruct((B, IN_CHANNELS), jnp.float32),
    )(xt)

    weights, indices = pl.pallas_call(
        _router_kernel,
        in_specs=[
            pl.BlockSpec((B, IN_CHANNELS), lambda: (0, 0)),
            pl.BlockSpec((B, IN_CHANNELS), lambda: (0, 0)),
            pl.BlockSpec((N_EXPERTS, IN_CHANNELS), lambda: (0, 0)),
            pl.BlockSpec((1, N_EXPERTS), lambda: (0, 0)),
            pl.BlockSpec((1, N_EXPERTS), lambda: (0, 0)),
        ],
        out_specs=[
            pl.BlockSpec((B, TOP_K), lambda: (0, 0)),
            pl.BlockSpec((B, TOP_K), lambda: (0, 0)),
        ],
        out_shape=[
            jax.ShapeDtypeStruct((B, TOP_K), x.dtype),
            jax.ShapeDtypeStruct((B, TOP_K), jnp.int32),
        ],
    )(sums_tc, sums_sc, W, b.reshape(1, N_EXPERTS), bias_buf.reshape(1, N_EXPERTS))
    return weights, indices


# speedup vs baseline: 1.3343x; 1.3343x over previous
import jax
import jax.numpy as jnp
from jax.experimental import pallas as pl
from jax.experimental.pallas import tpu as pltpu

IN_CHANNELS = 384
N_EXPERTS = 64
TOP_K = 8
ROUTE_SCALE = 1.0

B = 32
H = 56
W_SP = 56
SPATIAL = H * W_SP

BATCH_BLK = 8
H_BLK = 7
N_BATCH_BLKS = B // BATCH_BLK
N_H_BLKS = H // H_BLK


def _gate_kernel(x_ref, w_ref, b_ref, bias_ref, wout_ref, iout_ref, acc_ref):
    bi = pl.program_id(0)
    hi = pl.program_id(1)

    part = jnp.sum(x_ref[...], axis=(1, 2))
    rows = pl.ds(bi * BATCH_BLK, BATCH_BLK)

    @pl.when(hi == 0)
    def _init():
        acc_ref[rows, :] = part

    @pl.when(hi != 0)
    def _accum():
        acc_ref[rows, :] += part

    @pl.when((bi == N_BATCH_BLKS - 1) & (hi == N_H_BLKS - 1))
    def _epilogue():
        pooled = acc_ref[...] * (1.0 / SPATIAL)
        logits = jax.lax.dot_general(
            pooled,
            w_ref[...],
            (((1,), (1,)), ((), ())),
            preferred_element_type=jnp.float32,
        ) + b_ref[...]
        scores = jax.nn.sigmoid(logits)
        s = scores + bias_ref[...]

        iota = jax.lax.broadcasted_iota(jnp.int32, (B, N_EXPERTS), 1)
        idx_cols = []
        w_cols = []
        for _ in range(TOP_K):
            m = jnp.max(s, axis=1, keepdims=True)
            idx = jnp.min(
                jnp.where(s == m, iota, N_EXPERTS), axis=1, keepdims=True
            )
            onehot = iota == idx
            w = jnp.sum(jnp.where(onehot, scores, 0.0), axis=1, keepdims=True)
            idx_cols.append(idx)
            w_cols.append(w)
            s = jnp.where(onehot, -jnp.inf, s)
        indices = jnp.concatenate(idx_cols, axis=1)
        weights = jnp.concatenate(w_cols, axis=1)
        weights = weights / jnp.sum(weights, axis=1, keepdims=True)
        wout_ref[...] = weights * ROUTE_SCALE
        iout_ref[...] = indices


@jax.jit
def kernel(x, W, b, bias_buf):
    xt = x.transpose(0, 2, 3, 1)
    weights, indices = pl.pallas_call(
        _gate_kernel,
        grid=(N_BATCH_BLKS, N_H_BLKS),
        in_specs=[
            pl.BlockSpec(
                (BATCH_BLK, H_BLK, W_SP, IN_CHANNELS), lambda bi, hi: (bi, hi, 0, 0)
            ),
            pl.BlockSpec((N_EXPERTS, IN_CHANNELS), lambda bi, hi: (0, 0)),
            pl.BlockSpec((1, N_EXPERTS), lambda bi, hi: (0, 0)),
            pl.BlockSpec((1, N_EXPERTS), lambda bi, hi: (0, 0)),
        ],
        out_specs=[
            pl.BlockSpec((B, TOP_K), lambda bi, hi: (0, 0)),
            pl.BlockSpec((B, TOP_K), lambda bi, hi: (0, 0)),
        ],
        out_shape=[
            jax.ShapeDtypeStruct((B, TOP_K), x.dtype),
            jax.ShapeDtypeStruct((B, TOP_K), jnp.int32),
        ],
        scratch_shapes=[pltpu.VMEM((B, IN_CHANNELS), jnp.float32)],
    )(xt, W, b.reshape(1, N_EXPERTS), bias_buf.reshape(1, N_EXPERTS))
    return weights, indices


# speedup vs baseline: 1.3913x; 1.0428x over previous
import jax
import jax.numpy as jnp
from jax.experimental import pallas as pl
from jax.experimental.pallas import tpu as pltpu

IN_CHANNELS = 384
N_EXPERTS = 64
TOP_K = 8
ROUTE_SCALE = 1.0

B = 32
H = 56
W_SP = 56
SPATIAL = H * W_SP

BATCH_BLK = 8
H_BLK = 14
N_BATCH_BLKS = B // BATCH_BLK
N_H_BLKS = H // H_BLK


def _gate_kernel(x_ref, w_ref, b_ref, bias_ref, wout_ref, iout_ref, acc_ref):
    bi = pl.program_id(0)
    hi = pl.program_id(1)

    part = jnp.sum(x_ref[...], axis=(1, 2))
    rows = pl.ds(bi * BATCH_BLK, BATCH_BLK)

    @pl.when(hi == 0)
    def _init():
        acc_ref[rows, :] = part

    @pl.when(hi != 0)
    def _accum():
        acc_ref[rows, :] += part

    @pl.when((bi == N_BATCH_BLKS - 1) & (hi == N_H_BLKS - 1))
    def _epilogue():
        pooled = acc_ref[...] * (1.0 / SPATIAL)
        logits = jax.lax.dot_general(
            pooled,
            w_ref[...],
            (((1,), (1,)), ((), ())),
            preferred_element_type=jnp.float32,
        ) + b_ref[...]
        scores = jax.nn.sigmoid(logits)
        s = scores + bias_ref[...]

        s_l = jnp.broadcast_to(s[:, None, :], (B, N_EXPERTS, N_EXPERTS))
        s_s = jnp.broadcast_to(s[:, :, None], (B, N_EXPERTS, N_EXPERTS))
        i_l = jax.lax.broadcasted_iota(jnp.int32, (B, N_EXPERTS, N_EXPERTS), 2)
        i_s = jax.lax.broadcasted_iota(jnp.int32, (B, N_EXPERTS, N_EXPERTS), 1)
        gt = (s_s > s_l) | ((s_s == s_l) & (i_s < i_l))
        rank = jnp.sum(jnp.where(gt, 1, 0), axis=1)

        iota = jax.lax.broadcasted_iota(jnp.int32, (B, N_EXPERTS), 1)
        idx_cols = []
        w_cols = []
        for k in range(TOP_K):
            sel = rank == k
            idx_cols.append(
                jnp.sum(jnp.where(sel, iota, 0), axis=1, keepdims=True)
            )
            w_cols.append(
                jnp.sum(jnp.where(sel, scores, 0.0), axis=1, keepdims=True)
            )
        denom = jnp.sum(
            jnp.where(rank < TOP_K, scores, 0.0), axis=1, keepdims=True
        )
        indices = jnp.concatenate(idx_cols, axis=1)
        weights = jnp.concatenate(w_cols, axis=1) / denom
        wout_ref[...] = weights * ROUTE_SCALE
        iout_ref[...] = indices


@jax.jit
def kernel(x, W, b, bias_buf):
    xt = x.transpose(0, 2, 3, 1)
    weights, indices = pl.pallas_call(
        _gate_kernel,
        grid=(N_BATCH_BLKS, N_H_BLKS),
        in_specs=[
            pl.BlockSpec(
                (BATCH_BLK, H_BLK, W_SP, IN_CHANNELS), lambda bi, hi: (bi, hi, 0, 0)
            ),
            pl.BlockSpec((N_EXPERTS, IN_CHANNELS), lambda bi, hi: (0, 0)),
            pl.BlockSpec((1, N_EXPERTS), lambda bi, hi: (0, 0)),
            pl.BlockSpec((1, N_EXPERTS), lambda bi, hi: (0, 0)),
        ],
        out_specs=[
            pl.BlockSpec((B, TOP_K), lambda bi, hi: (0, 0)),
            pl.BlockSpec((B, TOP_K), lambda bi, hi: (0, 0)),
        ],
        out_shape=[
            jax.ShapeDtypeStruct((B, TOP_K), x.dtype),
            jax.ShapeDtypeStruct((B, TOP_K), jnp.int32),
        ],
        scratch_shapes=[pltpu.VMEM((B, IN_CHANNELS), jnp.float32)],
    )(xt, W, b.reshape(1, N_EXPERTS), bias_buf.reshape(1, N_EXPERTS))
    return weights, indices


# speedup vs baseline: 1.3991x; 1.0056x over previous
import jax
import jax.numpy as jnp
from jax.experimental import pallas as pl
from jax.experimental.pallas import tpu as pltpu

IN_CHANNELS = 384
N_EXPERTS = 64
TOP_K = 8
ROUTE_SCALE = 1.0

B = 32
H = 56
W_SP = 56
SPATIAL = H * W_SP

BATCH_BLK = 8
H_BLK = 14
N_BATCH_BLKS = B // BATCH_BLK
N_H_BLKS = H // H_BLK


def _gate_kernel(
    x_ref, w_ref, b_ref, bias_ref, wout_ref, iout_ref, acc_ref, sc_ref
):
    bi = pl.program_id(0)
    hi = pl.program_id(1)

    part = jnp.sum(x_ref[...], axis=(1, 2))
    rows = pl.ds(bi * BATCH_BLK, BATCH_BLK)

    @pl.when(hi == 0)
    def _init():
        acc_ref[...] = part

    @pl.when((hi != 0) & (hi != N_H_BLKS - 1))
    def _accum():
        acc_ref[...] += part

    @pl.when(hi == N_H_BLKS - 1)
    def _scores():
        pooled = (acc_ref[...] + part) * (1.0 / SPATIAL)
        logits = jax.lax.dot_general(
            pooled,
            w_ref[...],
            (((1,), (1,)), ((), ())),
            preferred_element_type=jnp.float32,
        ) + b_ref[...]
        sc_ref[rows, :] = jax.nn.sigmoid(logits)

    @pl.when((bi == N_BATCH_BLKS - 1) & (hi == N_H_BLKS - 1))
    def _epilogue():
        scores = sc_ref[...]
        s = scores + bias_ref[...]

        s_l = jnp.broadcast_to(s[:, None, :], (B, N_EXPERTS, N_EXPERTS))
        s_s = jnp.broadcast_to(s[:, :, None], (B, N_EXPERTS, N_EXPERTS))
        i_l = jax.lax.broadcasted_iota(jnp.int32, (B, N_EXPERTS, N_EXPERTS), 2)
        i_s = jax.lax.broadcasted_iota(jnp.int32, (B, N_EXPERTS, N_EXPERTS), 1)
        gt = (s_s > s_l) | ((s_s == s_l) & (i_s < i_l))
        rank = jnp.sum(jnp.where(gt, 1, 0), axis=1)

        iota = jax.lax.broadcasted_iota(jnp.int32, (B, N_EXPERTS), 1)
        idx_cols = []
        w_cols = []
        for k in range(TOP_K):
            sel = rank == k
            idx_cols.append(
                jnp.sum(jnp.where(sel, iota, 0), axis=1, keepdims=True)
            )
            w_cols.append(
                jnp.sum(jnp.where(sel, scores, 0.0), axis=1, keepdims=True)
            )
        denom = jnp.sum(
            jnp.where(rank < TOP_K, scores, 0.0), axis=1, keepdims=True
        )
        indices = jnp.concatenate(idx_cols, axis=1)
        weights = jnp.concatenate(w_cols, axis=1) / denom
        wout_ref[...] = weights * ROUTE_SCALE
        iout_ref[...] = indices


@jax.jit
def kernel(x, W, b, bias_buf):
    xt = x.transpose(0, 2, 3, 1)
    weights, indices = pl.pallas_call(
        _gate_kernel,
        grid=(N_BATCH_BLKS, N_H_BLKS),
        in_specs=[
            pl.BlockSpec(
                (BATCH_BLK, H_BLK, W_SP, IN_CHANNELS), lambda bi, hi: (bi, hi, 0, 0)
            ),
            pl.BlockSpec((N_EXPERTS, IN_CHANNELS), lambda bi, hi: (0, 0)),
            pl.BlockSpec((1, N_EXPERTS), lambda bi, hi: (0, 0)),
            pl.BlockSpec((1, N_EXPERTS), lambda bi, hi: (0, 0)),
        ],
        out_specs=[
            pl.BlockSpec((B, TOP_K), lambda bi, hi: (0, 0)),
            pl.BlockSpec((B, TOP_K), lambda bi, hi: (0, 0)),
        ],
        out_shape=[
            jax.ShapeDtypeStruct((B, TOP_K), x.dtype),
            jax.ShapeDtypeStruct((B, TOP_K), jnp.int32),
        ],
        scratch_shapes=[
            pltpu.VMEM((BATCH_BLK, IN_CHANNELS), jnp.float32),
            pltpu.VMEM((B, N_EXPERTS), jnp.float32),
        ],
    )(xt, W, b.reshape(1, N_EXPERTS), bias_buf.reshape(1, N_EXPERTS))
    return weights, indices
